# baseline (speedup 1.0000x reference)
import jax
import jax.numpy as jnp
from jax import lax
from jax.experimental import pallas as pl
from jax.experimental.pallas import tpu as pltpu

B, T, D = 16, 2048, 1024


def _body(seq_ref, x_ref, out_ref, sem):
    copies = []
    for b in range(B):
        t = lax.rem(seq_ref[b] + (T - 1), T)
        copies.append(pltpu.make_async_copy(x_ref.at[b, t], out_ref.at[b], sem))
    for c in copies:
        c.start()
    for c in copies:
        c.wait()


@jax.jit
def _take_last(x, seq):
    return pl.pallas_call(
        _body,
        in_specs=[
            pl.BlockSpec(memory_space=pltpu.SMEM),
            pl.BlockSpec(memory_space=pl.ANY),
        ],
        out_specs=pl.BlockSpec(memory_space=pltpu.VMEM),
        out_shape=jax.ShapeDtypeStruct((B, D), jnp.float32),
        scratch_shapes=[pltpu.SemaphoreType.DMA],
    )(seq, x)


def kernel(x, seq_len):
    return _take_last(x, seq_len.astype(jnp.int32))
